# baseline (speedup 1.0000x reference)
import jax
import jax.numpy as jnp
from jax import lax
from jax.experimental import pallas as pl
from jax.experimental.pallas import tpu as pltpu
from jax.experimental.pallas import tpu_sc as plsc
from jax._src.pallas import mpmd as _mpmd

N = 1_000_000
NIDX = 65536
NW = 32
ROWS = 16
RLEN = 128
ZBLK = 131_072
L = 16


def _zeros_body(o_ref):
    o_ref[...] = jnp.zeros((ZBLK,), jnp.float32)


_zeros_tc = pl.pallas_call(
    _zeros_body,
    out_shape=jax.ShapeDtypeStruct((N,), jnp.float32),
    grid=(pl.cdiv(N, ZBLK),),
    out_specs=pl.BlockSpec((ZBLK,), lambda i: (i,)),
)


def _scatter_body(zeros_hbm, idx_hbm, out_hbm, idx_v, ones_v, sem):
    del zeros_hbm
    w = lax.axis_index("s") * 2 + lax.axis_index("c")
    cp_idx = pltpu.async_copy(idx_hbm.at[w], idx_v, sem)
    ones = jnp.ones((L,), jnp.float32)
    for k in range(RLEN // L):
        ones_v[pl.ds(k * L, L)] = ones
    cp_idx.wait()
    cps = [
        pltpu.async_copy(ones_v, out_hbm.at[idx_v.at[j]], sem)
        for j in range(ROWS)
    ]
    for cp in cps:
        cp.wait()


_scatter_sc = _mpmd._mpmd_map(
    [(plsc.VectorSubcoreMesh(core_axis_name="c", subcore_axis_name="s"),
      _scatter_body)],
    [jax.ShapeDtypeStruct((N,), jnp.float32)],
    input_output_aliases={0: 0},
    scratch_types=[
        pltpu.VMEM((ROWS, RLEN), jnp.int32),
        pltpu.VMEM((RLEN,), jnp.float32),
        pltpu.SemaphoreType.DMA,
    ],
    compiler_params=pltpu.CompilerParams(needs_layout_passes=False),
)


def kernel(n_range, s, idx):
    del n_range, s
    idx3 = idx.astype(jnp.int32).reshape(NW, ROWS, RLEN)
    return (_scatter_sc(_zeros_tc(), idx3)[0],)


# speedup vs baseline: 1.1720x; 1.1720x over previous
import functools

import jax
import jax.numpy as jnp
from jax import lax
from jax.experimental import pallas as pl
from jax.experimental.pallas import tpu as pltpu
from jax.experimental.pallas import tpu_sc as plsc

N = 1_000_000
NIDX = 65536
HALF = N // 2
IMG = 524_288
DUMMY = HALF
ROWS = 32
RLEN = 128
ZB = 8192
WCH = 31256
WLAST = HALF - 15 * WCH
L = 16


@functools.partial(
    pl.kernel,
    mesh=plsc.VectorSubcoreMesh(core_axis_name="c", subcore_axis_name="s"),
    out_type=jax.ShapeDtypeStruct((N,), jnp.float32),
    scratch_types=[
        pltpu.VMEM((ROWS, RLEN), jnp.int32),
        pltpu.VMEM((ROWS + 1, RLEN), jnp.int32),
        pltpu.VMEM((ZB,), jnp.float32),
        pltpu.VMEM((RLEN,), jnp.float32),
        pltpu.VMEM_SHARED((IMG,), jnp.float32),
        pltpu.SemaphoreType.DMA,
        pltpu.SemaphoreType.DMA,
    ],
    compiler_params=pltpu.CompilerParams(needs_layout_passes=False),
)
def _scatter_ones(idx_hbm, out_hbm, idx_v, loc_v, zbuf, ones_v, img, sem1, sem2):
    c = lax.axis_index("c")
    s = lax.axis_index("s")
    lo = c * HALF

    cp_idx = pltpu.async_copy(idx_hbm.at[s], idx_v, sem1)

    zeros = jnp.zeros((L,), jnp.float32)

    def _fill_zeros(i, carry):
        zbuf[pl.ds(i * L, L)] = zeros
        return carry

    lax.fori_loop(0, ZB // L, _fill_zeros, 0)

    ones = jnp.ones((L,), jnp.float32)
    dummy = jnp.full((L,), DUMMY, jnp.int32)
    for k in range(RLEN // L):
        ones_v[pl.ds(k * L, L)] = ones
        loc_v[ROWS, pl.ds(k * L, L)] = dummy

    zcps = [
        pltpu.async_copy(zbuf, img.at[pl.ds(s * 4 * ZB + k * ZB, ZB)], sem2)
        for k in range(4)
    ]

    cp_idx.wait()
    hi = lo + HALF

    def _localize(i, carry):
        j = i // (RLEN // L)
        k = (i % (RLEN // L)) * L
        t = idx_v[j, pl.ds(k, L)]
        m = (t >= lo) & (t < hi)
        loc_v[j, pl.ds(k, L)] = jnp.where(m, t - lo, DUMMY)
        return carry

    lax.fori_loop(0, ROWS * RLEN // L, _localize, 0)

    for cp in zcps:
        cp.wait()
    plsc.subcore_barrier()

    for g in range(2):
        scps = [
            pltpu.async_copy(ones_v, img.at[loc_v.at[g * 16 + j]], sem2, add=True)
            for j in range(16)
        ]
        for cp in scps:
            cp.wait()

    pltpu.async_copy(ones_v, img.at[loc_v.at[ROWS]], sem2, add=True).wait()

    plsc.subcore_barrier()

    one = jnp.ones((L,), jnp.float32)

    def _wout(total):
        offs = [0, ZB, 2 * ZB, 3 * ZB]
        szs = [ZB, ZB, ZB, total - 3 * ZB]
        base = s * WCH
        for o, sz in zip(offs, szs):
            pltpu.sync_copy(img.at[pl.ds(base + o, sz)], zbuf.at[pl.ds(0, sz)])

            def _clamp(i, carry):
                zbuf[pl.ds(i * L, L)] = jnp.minimum(zbuf[pl.ds(i * L, L)], one)
                return carry

            lax.fori_loop(0, ZB // L, _clamp, 0)
            pltpu.sync_copy(zbuf.at[pl.ds(0, sz)],
                            out_hbm.at[pl.ds(lo + base + o, sz)])

    @pl.when(s < 15)
    def _():
        _wout(WCH)

    @pl.when(s == 15)
    def _():
        _wout(WLAST)


def kernel(n_range, s, idx):
    del n_range, s
    idx3 = idx.astype(jnp.int32).reshape(16, ROWS, RLEN)
    return (_scatter_ones(idx3),)


# speedup vs baseline: 2.6525x; 2.2632x over previous
import functools

import jax
import jax.numpy as jnp
from jax import lax
from jax.experimental import pallas as pl
from jax.experimental.pallas import tpu as pltpu
from jax.experimental.pallas import tpu_sc as plsc

N = 1_000_000
NIDX = 65536
NW = 32
CHUNK = 31360
LAST = N - 31 * CHUNK
L = 16


@functools.partial(
    pl.kernel,
    mesh=plsc.VectorSubcoreMesh(core_axis_name="c", subcore_axis_name="s"),
    out_type=jax.ShapeDtypeStruct((N,), jnp.float32),
    scratch_types=[
        pltpu.VMEM((NIDX,), jnp.int32),
        pltpu.VMEM((CHUNK,), jnp.float32),
        pltpu.SemaphoreType.DMA,
    ],
    compiler_params=pltpu.CompilerParams(needs_layout_passes=False),
)
def _scatter_ones(idx_hbm, out_hbm, idx_v, chunk_v, sem):
    wid = lax.axis_index("s") * 2 + lax.axis_index("c")
    base = wid * CHUNK

    cp = pltpu.async_copy(idx_hbm, idx_v, sem)

    zeros = jnp.zeros((L,), jnp.float32)

    @plsc.parallel_loop(0, CHUNK // L, unroll=8)
    def _zero_body(i):
        chunk_v[pl.ds(i * L, L)] = zeros

    cp.wait()

    ones = jnp.ones((L,), jnp.float32)

    @plsc.parallel_loop(0, NIDX // L, unroll=16)
    def _scan_body(j):
        loc = idx_v[pl.ds(j * L, L)] - base
        m = plsc.bitcast(loc, jnp.uint32) < jnp.uint32(CHUNK)
        plsc.store_scatter(chunk_v, [loc], ones, mask=m)

    @pl.when(wid < NW - 1)
    def _():
        pltpu.sync_copy(chunk_v.at[pl.ds(0, CHUNK)], out_hbm.at[pl.ds(base, CHUNK)])

    @pl.when(wid == NW - 1)
    def _():
        pltpu.sync_copy(chunk_v.at[pl.ds(0, LAST)], out_hbm.at[pl.ds(base, LAST)])


def kernel(n_range, s, idx):
    del n_range, s
    return (_scatter_ones(idx.astype(jnp.int32)),)


# speedup vs baseline: 2.7935x; 1.0532x over previous
import functools

import jax
import jax.numpy as jnp
from jax import lax
from jax.experimental import pallas as pl
from jax.experimental.pallas import tpu as pltpu
from jax.experimental.pallas import tpu_sc as plsc

N = 1_000_000
NIDX = 65536
NW = 32
CHUNK = 31360
LAST = N - 31 * CHUNK
L = 16


@functools.partial(
    pl.kernel,
    mesh=plsc.VectorSubcoreMesh(core_axis_name="c", subcore_axis_name="s"),
    out_type=jax.ShapeDtypeStruct((N,), jnp.float32),
    scratch_types=[
        pltpu.VMEM((NIDX,), jnp.int32),
        pltpu.VMEM((CHUNK,), jnp.float32),
        pltpu.SemaphoreType.DMA,
    ],
    compiler_params=pltpu.CompilerParams(needs_layout_passes=False),
)
def _scatter_ones(idx_hbm, out_hbm, idx_v, chunk_v, sem):
    wid = lax.axis_index("s") * 2 + lax.axis_index("c")
    base = wid * CHUNK

    ich = NIDX // NW
    cps = []
    for k in range(NW):
        off = ((wid + k) % NW) * ich
        cps.append(pltpu.async_copy(idx_hbm.at[pl.ds(off, ich)],
                                    idx_v.at[pl.ds(off, ich)], sem))

    zeros = jnp.zeros((L,), jnp.float32)

    @plsc.parallel_loop(0, CHUNK // L, unroll=8)
    def _zero_body(i):
        chunk_v[pl.ds(i * L, L)] = zeros

    for cp in cps:
        cp.wait()

    ones = jnp.ones((L,), jnp.float32)

    @plsc.parallel_loop(0, NIDX // L, unroll=16)
    def _scan_body(j):
        loc = idx_v[pl.ds(j * L, L)] - base
        m = plsc.bitcast(loc, jnp.uint32) < jnp.uint32(CHUNK)
        plsc.store_scatter(chunk_v, [loc], ones, mask=m)

    @pl.when(wid < NW - 1)
    def _():
        pltpu.sync_copy(chunk_v.at[pl.ds(0, CHUNK)], out_hbm.at[pl.ds(base, CHUNK)])

    @pl.when(wid == NW - 1)
    def _():
        pltpu.sync_copy(chunk_v.at[pl.ds(0, LAST)], out_hbm.at[pl.ds(base, LAST)])


def kernel(n_range, s, idx):
    del n_range, s
    return (_scatter_ones(idx.astype(jnp.int32)),)


# speedup vs baseline: 2.8506x; 1.0204x over previous
import functools

import jax
import jax.numpy as jnp
from jax import lax
from jax.experimental import pallas as pl
from jax.experimental.pallas import tpu as pltpu
from jax.experimental.pallas import tpu_sc as plsc

N = 1_000_000
NIDX = 65536
NW = 32
CHUNK = 31360
LAST = N - 31 * CHUNK
L = 16


@functools.partial(
    pl.kernel,
    mesh=plsc.VectorSubcoreMesh(core_axis_name="c", subcore_axis_name="s"),
    out_type=jax.ShapeDtypeStruct((N,), jnp.float32),
    scratch_types=[
        pltpu.VMEM((NIDX,), jnp.int32),
        pltpu.VMEM((CHUNK,), jnp.float32),
        [pltpu.SemaphoreType.DMA] * 8,
    ],
    compiler_params=pltpu.CompilerParams(needs_layout_passes=False),
)
def _scatter_ones(idx_hbm, out_hbm, idx_v, chunk_v, sems):
    wid = lax.axis_index("s") * 2 + lax.axis_index("c")
    base = wid * CHUNK

    nch = 8
    ich = NIDX // nch
    offs = [lax.rem(jnp.int32(wid // 4 + k), jnp.int32(nch)) * ich
            for k in range(nch)]
    cps = [pltpu.async_copy(idx_hbm.at[pl.ds(offs[k], ich)],
                            idx_v.at[pl.ds(offs[k], ich)], sems[k])
           for k in range(nch)]

    zeros = jnp.zeros((L,), jnp.float32)

    @plsc.parallel_loop(0, CHUNK // L, unroll=8)
    def _zero_body(i):
        chunk_v[pl.ds(i * L, L)] = zeros

    ones = jnp.ones((L,), jnp.float32)

    for k in range(nch):
        cps[k].wait()
        off_k = offs[k]

        @plsc.parallel_loop(0, ich // L, unroll=16)
        def _scan_body(j):
            loc = idx_v[pl.ds(off_k + j * L, L)] - base
            m = plsc.bitcast(loc, jnp.uint32) < jnp.uint32(CHUNK)
            plsc.store_scatter(chunk_v, [loc], ones, mask=m)

    @pl.when(wid < NW - 1)
    def _():
        pltpu.sync_copy(chunk_v.at[pl.ds(0, CHUNK)], out_hbm.at[pl.ds(base, CHUNK)])

    @pl.when(wid == NW - 1)
    def _():
        pltpu.sync_copy(chunk_v.at[pl.ds(0, LAST)], out_hbm.at[pl.ds(base, LAST)])


def kernel(n_range, s, idx):
    del n_range, s
    return (_scatter_ones(idx.astype(jnp.int32)),)


# speedup vs baseline: 3.0999x; 1.0875x over previous
import functools

import jax
import jax.numpy as jnp
from jax import lax
from jax.experimental import pallas as pl
from jax.experimental.pallas import tpu as pltpu
from jax.experimental.pallas import tpu_sc as plsc

N = 1_000_000
NIDX = 65536
NW = 16
CHUNK = 62720
LAST = N - 15 * CHUNK
L = 16


@functools.partial(
    pl.kernel,
    mesh=plsc.VectorSubcoreMesh(core_axis_name="c", subcore_axis_name="s",
                                num_cores=1),
    out_type=jax.ShapeDtypeStruct((N,), jnp.float32),
    scratch_types=[
        pltpu.VMEM((NIDX,), jnp.int32),
        pltpu.VMEM((CHUNK,), jnp.float32),
        [pltpu.SemaphoreType.DMA] * 8,
    ],
    compiler_params=pltpu.CompilerParams(needs_layout_passes=False),
)
def _scatter_ones(idx_hbm, out_hbm, idx_v, chunk_v, sems):
    wid = lax.axis_index("s")
    base = wid * CHUNK

    nch = 8
    ich = NIDX // nch
    offs = [lax.rem(jnp.int32(wid // 2 + k), jnp.int32(nch)) * ich
            for k in range(nch)]
    cps = [pltpu.async_copy(idx_hbm.at[pl.ds(offs[k], ich)],
                            idx_v.at[pl.ds(offs[k], ich)], sems[k])
           for k in range(nch)]

    zeros = jnp.zeros((L,), jnp.float32)

    @plsc.parallel_loop(0, CHUNK // L, unroll=8)
    def _zero_body(i):
        chunk_v[pl.ds(i * L, L)] = zeros

    ones = jnp.ones((L,), jnp.float32)

    for k in range(nch):
        cps[k].wait()
        off_k = offs[k]

        @plsc.parallel_loop(0, ich // L, unroll=16)
        def _scan_body(j):
            loc = idx_v[pl.ds(off_k + j * L, L)] - base
            m = plsc.bitcast(loc, jnp.uint32) < jnp.uint32(CHUNK)
            plsc.store_scatter(chunk_v, [loc], ones, mask=m)

    @pl.when(wid < NW - 1)
    def _():
        pltpu.sync_copy(chunk_v.at[pl.ds(0, CHUNK)], out_hbm.at[pl.ds(base, CHUNK)])

    @pl.when(wid == NW - 1)
    def _():
        pltpu.sync_copy(chunk_v.at[pl.ds(0, LAST)], out_hbm.at[pl.ds(base, LAST)])


def kernel(n_range, s, idx):
    del n_range, s
    return (_scatter_ones(idx.astype(jnp.int32)),)
